# Initial kernel scaffold; baseline (speedup 1.0000x reference)
import jax
import jax.numpy as jnp
from jax.experimental import pallas as pl


def kernel(data, table):
    raise NotImplementedError("write your pallas kernel here")


# speedup vs baseline: 7.3850x; 7.3850x over previous
import jax
import jax.numpy as jnp
from jax.experimental import pallas as pl
from jax.experimental.pallas import tpu as pltpu
from jax.experimental.pallas import tpu_sc as plsc

_WINDOW = 128


def kernel(data, table):
    batch, hist = data.shape
    vocab, d_model = table.shape
    num_indices = batch * hist

    idx = data.reshape(1, num_indices)
    mesh = plsc.VectorSubcoreMesh(core_axis_name="core",
                                  subcore_axis_name="subcore")

    @pl.kernel(
        out_type=jax.ShapeDtypeStruct((num_indices, d_model), table.dtype),
        mesh=mesh,
    )
    def gather_kernel(table_hbm, idx_hbm, out_hbm):
        def body(idx_vmem, out_vmem):
            pltpu.sync_copy(table_hbm.at[idx_vmem.at[0]], out_vmem)

        pltpu.emit_pipeline(
            body,
            grid=(num_indices // _WINDOW,),
            in_specs=[pl.BlockSpec((1, _WINDOW), index_map=lambda i: (0, i))],
            out_specs=[pl.BlockSpec((_WINDOW, d_model),
                                    index_map=lambda i: (i, 0))],
            core_axis_name=("core", "subcore"),
            dimension_semantics=(pltpu.PARALLEL,),
        )(idx_hbm, out_hbm)

    out = gather_kernel(table, idx)
    return out.reshape(batch, hist, d_model)


# speedup vs baseline: 9.0810x; 1.2297x over previous
import jax
import jax.numpy as jnp
from jax.experimental import pallas as pl
from jax.experimental.pallas import tpu as pltpu
from jax.experimental.pallas import tpu_sc as plsc

_WINDOW = 256


def kernel(data, table):
    batch, hist = data.shape
    vocab, d_model = table.shape
    num_indices = batch * hist

    idx = data.reshape(1, num_indices)
    mesh = plsc.VectorSubcoreMesh(core_axis_name="core",
                                  subcore_axis_name="subcore")

    @pl.kernel(
        out_type=jax.ShapeDtypeStruct((num_indices, d_model), table.dtype),
        mesh=mesh,
    )
    def gather_kernel(table_hbm, idx_hbm, out_hbm):
        def body(idx_vmem, out_vmem):
            pltpu.sync_copy(table_hbm.at[idx_vmem.at[0]], out_vmem)

        pltpu.emit_pipeline(
            body,
            grid=(num_indices // _WINDOW,),
            in_specs=[pl.BlockSpec((1, _WINDOW), index_map=lambda i: (0, i))],
            out_specs=[pl.BlockSpec((_WINDOW, d_model),
                                    index_map=lambda i: (i, 0))],
            core_axis_name=("core", "subcore"),
            dimension_semantics=(pltpu.PARALLEL,),
        )(idx_hbm, out_hbm)

    out = gather_kernel(table, idx)
    return out.reshape(batch, hist, d_model)


# speedup vs baseline: 9.0907x; 1.0011x over previous
import jax
import jax.numpy as jnp
from jax import lax
from jax.experimental import pallas as pl
from jax.experimental.pallas import tpu as pltpu
from jax.experimental.pallas import tpu_sc as plsc

_NB = 3
_W = 256
_NW = 32


def kernel(data, table):
    batch, hist = data.shape
    vocab, d_model = table.shape
    num_indices = batch * hist
    per_w = num_indices // _NW
    n_chunks = per_w // _W
    n_main = n_chunks - (n_chunks % _NB)

    idx = data.reshape(_NW, n_chunks, 1, _W)
    mesh = plsc.VectorSubcoreMesh(core_axis_name="core",
                                  subcore_axis_name="subcore")

    @pl.kernel(
        out_type=jax.ShapeDtypeStruct((num_indices, d_model), table.dtype),
        mesh=mesh,
        scratch_types=[
            pltpu.VMEM((_NB, 1, _W), jnp.int32),
            pltpu.VMEM((_NB, _W, d_model), jnp.float32),
            pltpu.SemaphoreType.DMA((_NB,)),
            pltpu.SemaphoreType.DMA((_NB,)),
            pltpu.SemaphoreType.DMA((_NB,)),
        ],
    )
    def gather_kernel(table_hbm, idx_hbm, out_hbm, idx_v, bufs, sem_i,
                      sem_g, sem_s):
        wid = lax.axis_index("core") * 16 + lax.axis_index("subcore")
        base = wid * per_w

        def start_idx(i, b):
            pltpu.make_async_copy(idx_hbm.at[wid, i], idx_v.at[b],
                                  sem_i.at[b]).start()

        def wait_idx(b):
            pltpu.make_async_copy(idx_hbm.at[wid, 0], idx_v.at[b],
                                  sem_i.at[b]).wait()

        def start_gather(b):
            pltpu.make_async_copy(table_hbm.at[idx_v.at[b, 0]], bufs.at[b],
                                  sem_g.at[b]).start()

        def wait_gather(b):
            pltpu.make_async_copy(table_hbm.at[idx_v.at[b, 0]], bufs.at[b],
                                  sem_g.at[b]).wait()

        def start_store(i, b):
            pltpu.make_async_copy(bufs.at[b],
                                  out_hbm.at[pl.ds(base + i * _W, _W)],
                                  sem_s.at[b]).start()

        def wait_store(i, b):
            pltpu.make_async_copy(bufs.at[b],
                                  out_hbm.at[pl.ds(base + i * _W, _W)],
                                  sem_s.at[b]).wait()

        for b in range(_NB):
            start_idx(b, b)

        @pl.loop(0, n_main, step=_NB)
        def _(g0):
            for b in range(_NB):

                @pl.when(g0 + b >= _NB)
                def _():
                    wait_store(g0 + b - _NB, b)

                wait_idx(b)
                start_gather(b)
            for b in range(_NB):
                wait_gather(b)
                start_store(g0 + b, b)

                @pl.when(g0 + b + _NB < n_chunks)
                def _():
                    start_idx(g0 + b + _NB, b)

        for i in range(n_main, n_chunks):
            b = i % _NB
            wait_store(i - _NB, b)
            wait_idx(b)
            start_gather(b)
            wait_gather(b)
            start_store(i, b)

        for i in range(n_chunks - _NB, n_chunks):
            wait_store(i, i % _NB)

    out = gather_kernel(table, idx)
    return out.reshape(batch, hist, d_model)


# speedup vs baseline: 9.0924x; 1.0002x over previous
import jax
import jax.numpy as jnp
from jax.experimental import pallas as pl
from jax.experimental.pallas import tpu as pltpu
from jax.experimental.pallas import tpu_sc as plsc

_WINDOW = 256


def kernel(data, table):
    batch, hist = data.shape
    vocab, d_model = table.shape
    num_indices = batch * hist

    idx = data.reshape(1, num_indices)
    mesh = plsc.VectorSubcoreMesh(core_axis_name="core",
                                  subcore_axis_name="subcore")

    @pl.kernel(
        out_type=jax.ShapeDtypeStruct((num_indices, d_model), table.dtype),
        mesh=mesh,
    )
    def gather_kernel(table_hbm, idx_hbm, out_hbm):
        def body(idx_vmem, out_vmem):
            pltpu.sync_copy(table_hbm.at[idx_vmem.at[0]], out_vmem)

        pltpu.emit_pipeline(
            body,
            grid=(num_indices // _WINDOW,),
            in_specs=[pl.BlockSpec((1, _WINDOW), index_map=lambda i: (0, i))],
            out_specs=[pl.BlockSpec((_WINDOW, d_model),
                                    index_map=lambda i: (i, 0))],
            core_axis_name=("core", "subcore"),
            dimension_semantics=(pltpu.PARALLEL,),
        )(idx_hbm, out_hbm)

    out = gather_kernel(table, idx)
    return out.reshape(batch, hist, d_model)
